# baseline (speedup 1.0000x reference)
import jax
import jax.numpy as jnp
from jax import lax
from jax.experimental import pallas as pl
from jax.experimental.pallas import tpu as pltpu
from jax.experimental.pallas import tpu_sc as plsc

_ROWS = 16384
_D = 2048
_NC = 2
_NS = 16
_NW = _NC * _NS
_RPW = _ROWS // _NW
_R = 16
_NCH = _RPW // _R
_NBUF = 3


def _sc_body(x_hbm, s_hbm, tbl_hbm, out_hbm, tbl_v, idx_v, xbuf,
             in_sem, out_sem):
    w = lax.axis_index("s") * _NC + lax.axis_index("c")
    base = w * _RPW

    pltpu.sync_copy(tbl_hbm, tbl_v)
    pltpu.sync_copy(s_hbm.at[pl.ds(base, _RPW)], idx_v)

    def start_in(c, bf):
        pltpu.async_copy(x_hbm.at[pl.ds(base + c * _R, _R)], xbuf.at[bf],
                         in_sem.at[bf])

    def wait_in(c, bf):
        pltpu.make_async_copy(x_hbm.at[pl.ds(base + c * _R, _R)],
                              xbuf.at[bf], in_sem.at[bf]).wait()

    def start_out(c, bf):
        pltpu.async_copy(xbuf.at[bf], out_hbm.at[pl.ds(base + c * _R, _R)],
                         out_sem.at[bf])

    def wait_out(c, bf):
        pltpu.make_async_copy(xbuf.at[bf],
                              out_hbm.at[pl.ds(base + c * _R, _R)],
                              out_sem.at[bf]).wait()

    for p in range(_NBUF - 1):
        start_in(p, p)

    def chunk_body(c, carry):
        bf = c % _NBUF
        nbf = (c + _NBUF - 1) % _NBUF
        wait_in(c, bf)

        @pl.when(c >= 1)
        def _():
            wait_out(c - 1, nbf)

        @pl.when(c + _NBUF - 1 < _NCH)
        def _():
            start_in(c + _NBUF - 1, nbf)

        s16 = idx_v[pl.ds(c * _R, 16)]
        for r in range(_R):
            sD = s16[r] * _D

            def col_body(jb, carry3, r=r, sD=sD):
                offs = [jb * 128 + k * 16 for k in range(8)]
                es = [tbl_v[pl.ds(sD + off, 16)] for off in offs]
                for off, e in zip(offs, es):
                    plsc.addupdate(xbuf.at[bf, r, pl.ds(off, 16)], e)
                return carry3

            lax.fori_loop(0, _D // 128, col_body, 0)

        start_out(c, bf)
        return carry

    lax.fori_loop(0, _NCH, chunk_body, 0)
    wait_out(_NCH - 1, (_NCH - 1) % _NBUF)


@jax.jit
def _sc_run(x2d, s1d, tbl):
    mesh = plsc.VectorSubcoreMesh(core_axis_name="c", subcore_axis_name="s",
                                  num_cores=_NC, num_subcores=_NS)
    f = pl.kernel(
        _sc_body,
        out_type=jax.ShapeDtypeStruct((_ROWS, _D), jnp.float32),
        mesh=mesh,
        scratch_types=[
            pltpu.VMEM((11 * _D,), jnp.float32),
            pltpu.VMEM((_RPW,), jnp.int32),
            pltpu.VMEM((_NBUF, _R, _D), jnp.float32),
            pltpu.SemaphoreType.DMA((_NBUF,)),
            pltpu.SemaphoreType.DMA((_NBUF,)),
        ],
        compiler_params=pltpu.CompilerParams(needs_layout_passes=False),
    )
    return f(x2d, s1d, tbl)


def kernel(x, scores, score_embeddings):
    b, n, d = x.shape
    x2d = x.reshape(b * n, d)
    s1d = scores.reshape(-1)
    out = _sc_run(x2d, s1d, score_embeddings.reshape(-1))
    return out.reshape(b, n, d)


# speedup vs baseline: 1.2101x; 1.2101x over previous
import jax
import jax.numpy as jnp
from jax import lax
from jax.experimental import pallas as pl
from jax.experimental.pallas import tpu as pltpu
from jax.experimental.pallas import tpu_sc as plsc

_ROWS = 16384
_D = 2048
_NC = 2
_NS = 16
_NW = _NC * _NS
_RPW = _ROWS // _NW
_R = 16
_NCH = _RPW // _R
_NBUF = 3


def _sc_body(x_hbm, s_hbm, tbl_hbm, out_hbm, tbl_v, idx_v, xbuf,
             in_sem, out_sem):
    w = lax.axis_index("s") * _NC + lax.axis_index("c")
    base = w * _RPW

    pltpu.sync_copy(tbl_hbm, tbl_v)
    pltpu.sync_copy(s_hbm.at[pl.ds(base, _RPW)], idx_v)

    def start_in(c, bf):
        pltpu.async_copy(x_hbm.at[pl.ds(base + c * _R, _R)], xbuf.at[bf],
                         in_sem.at[bf])

    def wait_in(c, bf):
        pltpu.make_async_copy(x_hbm.at[pl.ds(base + c * _R, _R)],
                              xbuf.at[bf], in_sem.at[bf]).wait()

    def start_out(c, bf):
        pltpu.async_copy(xbuf.at[bf], out_hbm.at[pl.ds(base + c * _R, _R)],
                         out_sem.at[bf])

    def wait_out(c, bf):
        pltpu.make_async_copy(xbuf.at[bf],
                              out_hbm.at[pl.ds(base + c * _R, _R)],
                              out_sem.at[bf]).wait()

    for p in range(_NBUF - 1):
        start_in(p, p)

    def chunk_body(c, carry):
        bf = c % _NBUF
        wait_in(c, bf)

        s16 = idx_v[pl.ds(c * _R, 16)]
        for r in range(_R):
            sD = s16[r] * _D

            def col_body(jb, carry3, r=r, sD=sD):
                offs = [jb * 128 + k * 16 for k in range(8)]
                es = [tbl_v[pl.ds(sD + off, 16)] for off in offs]
                for off, e in zip(offs, es):
                    plsc.addupdate(xbuf.at[bf, r, pl.ds(off, 16)], e)
                return carry3

            lax.fori_loop(0, _D // 128, col_body, 0)

        start_out(c, bf)

        nbf = (c + _NBUF - 1) % _NBUF

        @pl.when(c >= 1)
        def _():
            wait_out(c - 1, nbf)

        @pl.when(c + _NBUF - 1 < _NCH)
        def _():
            start_in(c + _NBUF - 1, nbf)

        return carry

    lax.fori_loop(0, _NCH, chunk_body, 0)
    wait_out(_NCH - 1, (_NCH - 1) % _NBUF)


@jax.jit
def _sc_run(x2d, s1d, tbl):
    mesh = plsc.VectorSubcoreMesh(core_axis_name="c", subcore_axis_name="s",
                                  num_cores=_NC, num_subcores=_NS)
    f = pl.kernel(
        _sc_body,
        out_type=jax.ShapeDtypeStruct((_ROWS, _D), jnp.float32),
        mesh=mesh,
        scratch_types=[
            pltpu.VMEM((11 * _D,), jnp.float32),
            pltpu.VMEM((_RPW,), jnp.int32),
            pltpu.VMEM((_NBUF, _R, _D), jnp.float32),
            pltpu.SemaphoreType.DMA((_NBUF,)),
            pltpu.SemaphoreType.DMA((_NBUF,)),
        ],
        compiler_params=pltpu.CompilerParams(needs_layout_passes=False),
    )
    return f(x2d, s1d, tbl)


def kernel(x, scores, score_embeddings):
    b, n, d = x.shape
    x2d = x.reshape(b * n, d)
    s1d = scores.reshape(-1)
    out = _sc_run(x2d, s1d, score_embeddings.reshape(-1))
    return out.reshape(b, n, d)


# speedup vs baseline: 1.2114x; 1.0010x over previous
import jax
import jax.numpy as jnp
from jax import lax
from jax.experimental import pallas as pl
from jax.experimental.pallas import tpu as pltpu
from jax.experimental.pallas import tpu_sc as plsc

_ROWS = 16384
_D = 2048
_NC = 2
_NS = 16
_NW = _NC * _NS
_RPW = _ROWS // _NW
_R = 8
_NCH = _RPW // _R
_NBUF = 6


def _sc_body(x_hbm, s_hbm, tbl_hbm, out_hbm, tbl_v, idx_v, xbuf,
             in_sem, out_sem):
    w = lax.axis_index("s") * _NC + lax.axis_index("c")
    base = w * _RPW

    pltpu.sync_copy(tbl_hbm, tbl_v)
    pltpu.sync_copy(s_hbm.at[pl.ds(base, _RPW)], idx_v)

    def start_in(c, bf):
        pltpu.async_copy(x_hbm.at[pl.ds(base + c * _R, _R)], xbuf.at[bf],
                         in_sem.at[bf])

    def wait_in(c, bf):
        pltpu.make_async_copy(x_hbm.at[pl.ds(base + c * _R, _R)],
                              xbuf.at[bf], in_sem.at[bf]).wait()

    def start_out(c, bf):
        pltpu.async_copy(xbuf.at[bf], out_hbm.at[pl.ds(base + c * _R, _R)],
                         out_sem.at[bf])

    def wait_out(c, bf):
        pltpu.make_async_copy(xbuf.at[bf],
                              out_hbm.at[pl.ds(base + c * _R, _R)],
                              out_sem.at[bf]).wait()

    for p in range(_NBUF - 1):
        start_in(p, p)

    def chunk_body(c, carry):
        bf = c % _NBUF
        wait_in(c, bf)

        s16 = idx_v[pl.ds((c // 2) * 16, 16)]
        odd = (c % 2) == 1
        for r in range(_R):
            sD = jnp.where(odd, s16[r + _R], s16[r]) * _D

            def col_body(jb, carry3, r=r, sD=sD):
                offs = [jb * 128 + k * 16 for k in range(8)]
                es = [tbl_v[pl.ds(sD + off, 16)] for off in offs]
                for off, e in zip(offs, es):
                    plsc.addupdate(xbuf.at[bf, r, pl.ds(off, 16)], e)
                return carry3

            lax.fori_loop(0, _D // 128, col_body, 0)

        start_out(c, bf)

        nbf = (c + _NBUF - 1) % _NBUF

        @pl.when(c >= 1)
        def _():
            wait_out(c - 1, nbf)

        @pl.when(c + _NBUF - 1 < _NCH)
        def _():
            start_in(c + _NBUF - 1, nbf)

        return carry

    lax.fori_loop(0, _NCH, chunk_body, 0)
    wait_out(_NCH - 1, (_NCH - 1) % _NBUF)


@jax.jit
def _sc_run(x2d, s1d, tbl):
    mesh = plsc.VectorSubcoreMesh(core_axis_name="c", subcore_axis_name="s",
                                  num_cores=_NC, num_subcores=_NS)
    f = pl.kernel(
        _sc_body,
        out_type=jax.ShapeDtypeStruct((_ROWS, _D), jnp.float32),
        mesh=mesh,
        scratch_types=[
            pltpu.VMEM((11 * _D,), jnp.float32),
            pltpu.VMEM((_RPW,), jnp.int32),
            pltpu.VMEM((_NBUF, _R, _D), jnp.float32),
            pltpu.SemaphoreType.DMA((_NBUF,)),
            pltpu.SemaphoreType.DMA((_NBUF,)),
        ],
        compiler_params=pltpu.CompilerParams(needs_layout_passes=False),
    )
    return f(x2d, s1d, tbl)


def kernel(x, scores, score_embeddings):
    b, n, d = x.shape
    x2d = x.reshape(b * n, d)
    s1d = scores.reshape(-1)
    out = _sc_run(x2d, s1d, score_embeddings.reshape(-1))
    return out.reshape(b, n, d)
